# Initial kernel scaffold; baseline (speedup 1.0000x reference)
import jax
import jax.numpy as jnp
from jax.experimental import pallas as pl


def kernel(means2d, conics, colors, opacities, depths):
    raise NotImplementedError("write your pallas kernel here")


# speedup vs baseline: 6.7145x; 6.7145x over previous
import functools

import jax
import jax.numpy as jnp
from jax import lax
from jax.experimental import pallas as pl
from jax.experimental.pallas import tpu as pltpu
from jax.experimental.pallas import tpu_sc as plsc

_H = 64
_W = 64
_FRONT_K = 8
_ALPHA_THR = 1.0 / 255.0
_G = 4096
_NPARAM = 16
_CHUNK = 16
_NQ = _W // 16


def _raster_body(params_hbm, out_hbm, pv, rowbuf, st, doneref):
    wid = lax.axis_index("s") * 2 + lax.axis_index("c")
    pltpu.sync_copy(params_hbm, pv)

    iota = lax.iota(jnp.int32, 16).astype(jnp.float32)
    ones = jnp.ones((16,), jnp.float32)
    zeros = jnp.zeros((16,), jnp.float32)
    px = [iota + (q * 16 + 0.5) for q in range(_NQ)]
    _T, _CNT, _AR, _AG, _AB = (0, _NQ, 2 * _NQ, 3 * _NQ, 4 * _NQ)

    for r2 in range(2):
        row = wid * 2 + r2
        py = row.astype(jnp.float32) + 0.5

        for q in range(_NQ):
            st[_T + q] = ones
            st[_CNT + q] = zeros
            st[_AR + q] = zeros
            st[_AG + q] = zeros
            st[_AB + q] = zeros
        doneref[0] = jnp.int32(0)

        def chunk_body(ci, carry):
            @pl.when(doneref[0] == 0)
            def _():
                T = [st[_T + q] for q in range(_NQ)]
                cnt = [st[_CNT + q] for q in range(_NQ)]
                ar = [st[_AR + q] for q in range(_NQ)]
                ag = [st[_AG + q] for q in range(_NQ)]
                ab = [st[_AB + q] for q in range(_NQ)]
                for u in range(_CHUNK):
                    g = ci * _CHUNK + u
                    prow = pv[pl.ds(g * _NPARAM, 16)]
                    mx = prow[0]
                    my = prow[1]
                    ca = prow[2]
                    cb = prow[3]
                    cc = prow[4]
                    op = prow[5]
                    colr = prow[6]
                    colg = prow[7]
                    colb = prow[8]
                    dy = py - my
                    cdy2 = 0.5 * cc * dy * dy
                    bdy = cb * dy
                    for q in range(_NQ):
                        dx = px[q] - mx
                        sigma = (0.5 * ca) * dx * dx + bdy * dx + cdy2
                        sigma = jnp.maximum(sigma, 0.0)
                        alpha = jnp.minimum(op * jnp.exp(-sigma), 0.999)
                        keep = jnp.logical_and(alpha >= _ALPHA_THR,
                                               cnt[q] < float(_FRONT_K))
                        ae = jnp.where(keep, alpha, 0.0)
                        w = ae * T[q]
                        ar[q] = ar[q] + w * colr
                        ag[q] = ag[q] + w * colg
                        ab[q] = ab[q] + w * colb
                        T[q] = T[q] * (1.0 - ae)
                        cnt[q] = cnt[q] + jnp.where(keep, 1.0, 0.0)
                for q in range(_NQ):
                    st[_T + q] = T[q]
                    st[_CNT + q] = cnt[q]
                    st[_AR + q] = ar[q]
                    st[_AG + q] = ag[q]
                    st[_AB + q] = ab[q]
                m = jnp.minimum(jnp.minimum(cnt[0], cnt[1]),
                                jnp.minimum(cnt[2], cnt[3]))
                done = jnp.min(m) >= float(_FRONT_K)
                doneref[0] = done.astype(jnp.int32)
            return carry

        lax.fori_loop(0, _G // _CHUNK, chunk_body, jnp.int32(0))

        for q in range(_NQ):
            rowbuf[0, pl.ds(q * 16, 16)] = st[_AR + q]
            rowbuf[1, pl.ds(q * 16, 16)] = st[_AG + q]
            rowbuf[2, pl.ds(q * 16, 16)] = st[_AB + q]
        for cpl in range(3):
            pltpu.sync_copy(rowbuf.at[cpl], out_hbm.at[cpl, row])


_raster = functools.partial(
    pl.kernel,
    out_type=jax.ShapeDtypeStruct((3, _H, _W), jnp.float32),
    scratch_types=[
        pltpu.VMEM((_G * _NPARAM,), jnp.float32),
        pltpu.VMEM((3, _W), jnp.float32),
        pltpu.VMEM((5 * _NQ, 16), jnp.float32),
        pltpu.SMEM((1,), jnp.int32),
    ],
    mesh=plsc.VectorSubcoreMesh(core_axis_name="c", subcore_axis_name="s"),
    compiler_params=pltpu.CompilerParams(needs_layout_passes=False),
)(_raster_body)


def kernel(means2d, conics, colors, opacities, depths):
    d = depths[0]
    perm = jnp.argsort(d, stable=True)
    m = means2d[0][perm]
    cn = conics[0][perm]
    col = colors[0][perm]
    op = opacities[0][perm]
    params = jnp.concatenate(
        [m, cn, op[:, None], col,
         jnp.zeros((_G, _NPARAM - 9), jnp.float32)], axis=1).reshape(-1)
    img = _raster(params)
    return jnp.transpose(img, (1, 2, 0))[None]


# speedup vs baseline: 13.1955x; 1.9652x over previous
import functools

import jax
import jax.numpy as jnp
from jax import lax
from jax.experimental import pallas as pl
from jax.experimental.pallas import tpu as pltpu
from jax.experimental.pallas import tpu_sc as plsc

_H = 64
_W = 64
_FRONT_K = 8
_ALPHA_THR = 1.0 / 255.0
_G = 4096
_NPARAM = 16
_CH = 128
_SUB = 16
_NQ = _W // 16


def _raster_body(params_hbm, perm_hbm, out_hbm,
                 permv, buf, st, rowbuf, doneref, sem):
    wid = lax.axis_index("s") * 2 + lax.axis_index("c")
    pltpu.sync_copy(perm_hbm, permv)

    iota = lax.iota(jnp.int32, 16).astype(jnp.float32)
    ones = jnp.ones((16,), jnp.float32)
    zeros = jnp.zeros((16,), jnp.float32)
    px = [iota + (q * 16 + 0.5) for q in range(_NQ)]
    def _sl(r, kind, q):
        return r * 5 * _NQ + kind * _NQ + q

    for r in range(2):
        for q in range(_NQ):
            st[_sl(r, 0, q)] = ones
            for k in range(1, 5):
                st[_sl(r, k, q)] = zeros
    doneref[0] = jnp.int32(0)

    pltpu.async_copy(params_hbm.at[permv.at[pl.ds(0, _CH)]],
                     buf.at[0], sem)

    def chunk_body(ci, carry):
        par = lax.rem(ci, 2)
        nci = ci + 1

        @pl.when(doneref[0] == 0)
        def _():
            pltpu.make_async_copy(params_hbm.at[pl.ds(0, _CH)],
                                  buf.at[par], sem).wait()

            @pl.when(nci < _G // _CH)
            def _():
                pltpu.async_copy(
                    params_hbm.at[permv.at[pl.ds(nci * _CH, _CH)]],
                    buf.at[1 - par], sem)

            def sub_body(s, scarry):
                @pl.when(doneref[0] == 0)
                def _():
                    mins = []
                    for r in range(2):
                        row = wid * 2 + r
                        py = row.astype(jnp.float32) + 0.5
                        T = [st[_sl(r, 0, q)] for q in range(_NQ)]
                        cnt = [st[_sl(r, 1, q)] for q in range(_NQ)]
                        ar = [st[_sl(r, 2, q)] for q in range(_NQ)]
                        ag = [st[_sl(r, 3, q)] for q in range(_NQ)]
                        ab = [st[_sl(r, 4, q)] for q in range(_NQ)]
                        for u in range(_SUB):
                            prow = buf[par, s * _SUB + u]
                            mx = prow[0]
                            my = prow[1]
                            ca = prow[2]
                            cb = prow[3]
                            cc = prow[4]
                            op = prow[5]
                            colr = prow[6]
                            colg = prow[7]
                            colb = prow[8]
                            dy = py - my
                            cdy2 = 0.5 * cc * dy * dy
                            bdy = cb * dy
                            ha = 0.5 * ca
                            for q in range(_NQ):
                                dx = px[q] - mx
                                sigma = ha * dx * dx + bdy * dx + cdy2
                                sigma = jnp.maximum(sigma, 0.0)
                                alpha = jnp.minimum(op * jnp.exp(-sigma),
                                                    0.999)
                                keep = jnp.logical_and(
                                    alpha >= _ALPHA_THR,
                                    cnt[q] < float(_FRONT_K))
                                ae = jnp.where(keep, alpha, 0.0)
                                w = ae * T[q]
                                ar[q] = ar[q] + w * colr
                                ag[q] = ag[q] + w * colg
                                ab[q] = ab[q] + w * colb
                                T[q] = T[q] * (1.0 - ae)
                                cnt[q] = cnt[q] + jnp.where(keep, 1.0, 0.0)
                        for q in range(_NQ):
                            st[_sl(r, 0, q)] = T[q]
                            st[_sl(r, 1, q)] = cnt[q]
                            st[_sl(r, 2, q)] = ar[q]
                            st[_sl(r, 3, q)] = ag[q]
                            st[_sl(r, 4, q)] = ab[q]
                        mins.append(jnp.minimum(
                            jnp.minimum(cnt[0], cnt[1]),
                            jnp.minimum(cnt[2], cnt[3])))
                    m = jnp.min(jnp.minimum(mins[0], mins[1]))
                    doneref[0] = (m >= float(_FRONT_K)).astype(jnp.int32)
                return scarry

            lax.fori_loop(0, _CH // _SUB, sub_body, jnp.int32(0))

            @pl.when(jnp.logical_and(doneref[0] == 1, nci < _G // _CH))
            def _():
                pltpu.make_async_copy(params_hbm.at[pl.ds(0, _CH)],
                                      buf.at[1 - par], sem).wait()

        return carry

    lax.fori_loop(0, _G // _CH, chunk_body, jnp.int32(0))

    for r in range(2):
        for q in range(_NQ):
            rowbuf[0, r, pl.ds(q * 16, 16)] = st[_sl(r, 2, q)]
            rowbuf[1, r, pl.ds(q * 16, 16)] = st[_sl(r, 3, q)]
            rowbuf[2, r, pl.ds(q * 16, 16)] = st[_sl(r, 4, q)]
    pltpu.sync_copy(rowbuf, out_hbm.at[:, pl.ds(2 * wid, 2), :])


_raster = functools.partial(
    pl.kernel,
    out_type=jax.ShapeDtypeStruct((3, _H, _W), jnp.float32),
    scratch_types=[
        pltpu.VMEM((_G,), jnp.int32),
        pltpu.VMEM((2, _CH, _NPARAM), jnp.float32),
        pltpu.VMEM((2 * 5 * _NQ, 16), jnp.float32),
        pltpu.VMEM((3, 2, _W), jnp.float32),
        pltpu.SMEM((1,), jnp.int32),
        pltpu.SemaphoreType.DMA,
    ],
    mesh=plsc.VectorSubcoreMesh(core_axis_name="c", subcore_axis_name="s"),
    compiler_params=pltpu.CompilerParams(needs_layout_passes=False, use_tc_tiling_on_sc=False),
)(_raster_body)


def kernel(means2d, conics, colors, opacities, depths):
    d = depths[0]
    perm = jnp.argsort(d, stable=True).astype(jnp.int32)
    params = jnp.concatenate(
        [means2d[0], conics[0], opacities[0][:, None], colors[0],
         jnp.zeros((_G, _NPARAM - 9), jnp.float32)], axis=1)
    img = _raster(params, perm)
    return jnp.transpose(img, (1, 2, 0))[None]


# speedup vs baseline: 13.4013x; 1.0156x over previous
import functools

import jax
import jax.numpy as jnp
from jax import lax
from jax.experimental import pallas as pl
from jax.experimental.pallas import tpu as pltpu
from jax.experimental.pallas import tpu_sc as plsc

_H = 64
_W = 64
_FRONT_K = 8
_ALPHA_THR = 1.0 / 255.0
_G = 4096
_NSOA = 9
_CH = 128
_SUB = 16
_NQ = _W // 16


def _raster_body(*refs):
    ins = refs[:_NSOA]
    out_hbm = refs[_NSOA]
    buf, st, rowbuf, doneref, sem = refs[_NSOA + 1:]

    wid = lax.axis_index("s") * 2 + lax.axis_index("c")

    iota = lax.iota(jnp.int32, 16).astype(jnp.float32)
    ones = jnp.ones((16,), jnp.float32)
    zeros = jnp.zeros((16,), jnp.float32)
    px = [iota + (q * 16 + 0.5) for q in range(_NQ)]
    def _sl(r, kind, q):
        return r * 5 * _NQ + kind * _NQ + q

    for r in range(2):
        for q in range(_NQ):
            st[_sl(r, 0, q)] = ones
            for k in range(1, 5):
                st[_sl(r, k, q)] = zeros
    doneref[0] = jnp.int32(0)

    def _fetch(ci, par):
        for k in range(_NSOA):
            pltpu.async_copy(ins[k].at[pl.ds(ci * _CH, _CH)],
                             buf.at[par, k], sem)

    def _wait_fetch(par):
        for k in range(_NSOA):
            pltpu.make_async_copy(ins[k].at[pl.ds(0, _CH)],
                                  buf.at[par, k], sem).wait()

    _fetch(0, 0)

    def chunk_body(ci, carry):
        par = lax.rem(ci, 2)
        nci = ci + 1

        @pl.when(doneref[0] == 0)
        def _():
            _wait_fetch(par)

            @pl.when(nci < _G // _CH)
            def _():
                _fetch(nci, 1 - par)

            def sub_body(s, scarry):
                @pl.when(doneref[0] == 0)
                def _():
                    sv = [buf[par, k, pl.ds(s * _SUB, _SUB)]
                          for k in range(_NSOA)]
                    mins = []
                    for r in range(2):
                        row = wid * 2 + r
                        py = row.astype(jnp.float32) + 0.5
                        T = [st[_sl(r, 0, q)] for q in range(_NQ)]
                        cnt = [st[_sl(r, 1, q)] for q in range(_NQ)]
                        ar = [st[_sl(r, 2, q)] for q in range(_NQ)]
                        ag = [st[_sl(r, 3, q)] for q in range(_NQ)]
                        ab = [st[_sl(r, 4, q)] for q in range(_NQ)]
                        for u in range(_SUB):
                            mx = sv[0][u]
                            my = sv[1][u]
                            ca = sv[2][u]
                            cb = sv[3][u]
                            cc = sv[4][u]
                            op = sv[5][u]
                            colr = sv[6][u]
                            colg = sv[7][u]
                            colb = sv[8][u]
                            dy = py - my
                            cdy2 = 0.5 * cc * dy * dy
                            bdy = cb * dy
                            ha = 0.5 * ca
                            for q in range(_NQ):
                                dx = px[q] - mx
                                sigma = ha * dx * dx + bdy * dx + cdy2
                                sigma = jnp.maximum(sigma, 0.0)
                                alpha = jnp.minimum(op * jnp.exp(-sigma),
                                                    0.999)
                                keep = jnp.logical_and(
                                    alpha >= _ALPHA_THR,
                                    cnt[q] < float(_FRONT_K))
                                ae = jnp.where(keep, alpha, 0.0)
                                w = ae * T[q]
                                ar[q] = ar[q] + w * colr
                                ag[q] = ag[q] + w * colg
                                ab[q] = ab[q] + w * colb
                                T[q] = T[q] * (1.0 - ae)
                                cnt[q] = cnt[q] + jnp.where(keep, 1.0, 0.0)
                        for q in range(_NQ):
                            st[_sl(r, 0, q)] = T[q]
                            st[_sl(r, 1, q)] = cnt[q]
                            st[_sl(r, 2, q)] = ar[q]
                            st[_sl(r, 3, q)] = ag[q]
                            st[_sl(r, 4, q)] = ab[q]
                        mins.append(jnp.minimum(
                            jnp.minimum(cnt[0], cnt[1]),
                            jnp.minimum(cnt[2], cnt[3])))
                    m = jnp.min(jnp.minimum(mins[0], mins[1]))
                    doneref[0] = (m >= float(_FRONT_K)).astype(jnp.int32)
                return scarry

            lax.fori_loop(0, _CH // _SUB, sub_body, jnp.int32(0))

            @pl.when(jnp.logical_and(doneref[0] == 1, nci < _G // _CH))
            def _():
                _wait_fetch(1 - par)

        return carry

    lax.fori_loop(0, _G // _CH, chunk_body, jnp.int32(0))

    for r in range(2):
        for q in range(_NQ):
            rowbuf[0, r, pl.ds(q * 16, 16)] = st[_sl(r, 2, q)]
            rowbuf[1, r, pl.ds(q * 16, 16)] = st[_sl(r, 3, q)]
            rowbuf[2, r, pl.ds(q * 16, 16)] = st[_sl(r, 4, q)]
    pltpu.sync_copy(rowbuf, out_hbm.at[:, pl.ds(2 * wid, 2), :])


_raster = functools.partial(
    pl.kernel,
    out_type=jax.ShapeDtypeStruct((3, _H, _W), jnp.float32),
    scratch_types=[
        pltpu.VMEM((2, _NSOA, _CH), jnp.float32),
        pltpu.VMEM((2 * 5 * _NQ, 16), jnp.float32),
        pltpu.VMEM((3, 2, _W), jnp.float32),
        pltpu.SMEM((1,), jnp.int32),
        pltpu.SemaphoreType.DMA,
    ],
    mesh=plsc.VectorSubcoreMesh(core_axis_name="c", subcore_axis_name="s"),
    compiler_params=pltpu.CompilerParams(needs_layout_passes=False,
                                         use_tc_tiling_on_sc=False),
)(_raster_body)


def kernel(means2d, conics, colors, opacities, depths):
    srt = lax.sort(
        (depths[0], means2d[0, :, 0], means2d[0, :, 1],
         conics[0, :, 0], conics[0, :, 1], conics[0, :, 2],
         opacities[0], colors[0, :, 0], colors[0, :, 1], colors[0, :, 2]),
        dimension=0, is_stable=True, num_keys=1)
    img = _raster(*srt[1:])
    return jnp.transpose(img, (1, 2, 0))[None]
